# baseline (speedup 1.0000x reference)
import functools

import jax
import jax.numpy as jnp
from jax import lax
from jax.experimental import pallas as pl
from jax.experimental.pallas import tpu as pltpu
from jax.experimental.pallas import tpu_sc as plsc

N = 10000
E = 320000
NP = 10240
NC = 2
NS = 16
EK = 100
NCH = 100
RPT = NP // NS
BR = 512

def _mesh():
    return plsc.VectorSubcoreMesh(
        core_axis_name="c", subcore_axis_name="s",
        num_cores=NC, num_subcores=NS)


@functools.cache
def _make_deg():
    return functools.partial(
        pl.kernel,
        out_type=jax.ShapeDtypeStruct((NC, NP), jnp.float32),
        mesh=_mesh(),
        scratch_types=[
            pltpu.VMEM((NCH, EK), jnp.int32),
            pltpu.VMEM((EK,), jnp.float32),
            pltpu.VMEM((RPT,), jnp.float32),
            pltpu.VMEM_SHARED((NP,), jnp.float32),
        ],
    )(_deg_body)


def _deg_body(dst_hbm, deg_hbm, didx, ones, dbuf, dacc):
    cid = lax.axis_index("c")
    tid = lax.axis_index("s")
    wid = tid * NC + cid

    def fill(i, _):
        dbuf[pl.ds(i * 16, 16)] = jnp.zeros((16,), jnp.float32)
        return 0
    lax.fori_loop(0, RPT // 16, fill, 0)

    def fill1(i, _):
        ones[pl.ds(i * 16, 16)] = jnp.ones((16,), jnp.float32)
        return 0
    lax.fori_loop(0, EK // 16, fill1, 0)

    pltpu.sync_copy(dst_hbm.at[wid], didx)
    pltpu.sync_copy(dbuf, dacc.at[pl.ds(tid * RPT, RPT)])
    plsc.subcore_barrier()

    def body(j, _):
        pltpu.sync_copy(ones, dacc.at[didx.at[j]], add=True)
        return 0
    lax.fori_loop(0, NCH, body, 0)
    plsc.subcore_barrier()

    pltpu.sync_copy(dacc.at[pl.ds(tid * RPT, RPT)], dbuf)
    pltpu.sync_copy(dbuf, deg_hbm.at[cid, pl.ds(tid * RPT, RPT)])


EPAD = 327680
APR = 80
ACH = 64
ANCH = 160
NBUF = 3


@functools.cache
def _make_agg():
    return functools.partial(
        pl.kernel,
        out_type=jax.ShapeDtypeStruct((NC, NP, 128), jnp.float32),
        mesh=_mesh(),
        scratch_types=[
            pltpu.VMEM((APR, 128), jnp.int32),
            [pltpu.VMEM((ACH,), jnp.int32) for _ in range(NBUF)],
            [pltpu.VMEM((ACH,), jnp.int32) for _ in range(NBUF)],
            [pltpu.VMEM((ACH, 128), jnp.float32) for _ in range(NBUF)],
            [pltpu.SemaphoreType.DMA for _ in range(NBUF)],
            [pltpu.SemaphoreType.DMA for _ in range(NBUF)],
            pltpu.VMEM_SHARED((NP, 128), jnp.float32),
        ],
    )(_agg_body)


def _agg_body(u_hbm, pk_hbm, out_hbm,
              pkbuf, sbufs, dbufs, rows, gsem, ssem, acc):
    cid = lax.axis_index("c")
    tid = lax.axis_index("s")
    wid = tid * NC + cid

    pltpu.sync_copy(pk_hbm.at[wid], pkbuf)

    def unpack(c, t):
        row = c // 2
        base = (c % 2) * 64
        for k in range(4):
            v = pkbuf[row, pl.ds(base + k * 16, 16)]
            sbufs[t][pl.ds(k * 16, 16)] = v & 0xFFFF
            dbufs[t][pl.ds(k * 16, 16)] = v >> 16

    def gather_start(j, b):
        pltpu.async_copy(u_hbm.at[sbufs[b]], rows[b], gsem[b])

    def gather_wait(b):
        pltpu.make_async_copy(u_hbm.at[sbufs[b]], rows[b], gsem[b]).wait()

    def scatter_start(j, b):
        pltpu.async_copy(rows[b], acc.at[dbufs[b]], ssem[b], add=True)

    def scatter_wait(b):
        pltpu.make_async_copy(rows[b], acc.at[dbufs[b]], ssem[b]).wait()

    def fill(i, _):
        rows[0][i // 8, pl.ds((i % 8) * 16, 16)] = jnp.zeros((16,), jnp.float32)
        return 0
    lax.fori_loop(0, ACH * 8, fill, 0)
    r0 = tid * RPT
    for k in range(RPT // ACH):
        pltpu.sync_copy(rows[0], acc.at[pl.ds(r0 + k * ACH, ACH)])
    unpack(0, 0)
    unpack(1, 1)
    gather_start(0, 0)
    gather_start(1, 1)
    plsc.subcore_barrier()

    gather_wait(0)
    scatter_start(0, 0)
    unpack(2, 2)
    gather_start(2, 2)
    gather_wait(1)
    scatter_start(1, 1)
    scatter_wait(0)
    unpack(3, 0)
    gather_start(3, 0)
    gather_wait(2)
    scatter_start(2, 2)
    scatter_wait(1)
    unpack(4, 1)
    gather_start(4, 1)

    def body(o, _):
        for b in range(NBUF):
            j = o * NBUF + b
            gather_wait(b)
            scatter_start(j, b)
            b2 = (b + 2) % NBUF
            scatter_wait(b2)
            unpack(j + 2, b2)
            gather_start(j + 2, b2)
        return 0
    lax.fori_loop(1, (ANCH - 4) // NBUF, body, 0)

    for j in range(ANCH - 4, ANCH):
        b = j % NBUF
        gather_wait(b)
        scatter_start(j, b)
        if j + 2 < ANCH:
            scatter_wait((j + 2) % NBUF)
            unpack(j + 2, (j + 2) % NBUF)
            gather_start(j + 2, (j + 2) % NBUF)
    scatter_wait((ANCH - 3) % NBUF)
    scatter_wait((ANCH - 2) % NBUF)
    scatter_wait((ANCH - 1) % NBUF)
    plsc.subcore_barrier()

    for k in range(RPT // ACH):
        pltpu.sync_copy(acc.at[pl.ds(r0 + k * ACH, ACH)], rows[0])
        pltpu.sync_copy(rows[0], out_hbm.at[cid, pl.ds(r0 + k * ACH, ACH)])


def _scale_body(x_ref, deg_ref, u_ref, dis_ref):
    deg = jnp.sum(deg_ref[...], axis=0)
    dis = lax.rsqrt(deg + 1.0)
    dis_ref[...] = dis
    u_ref[...] = x_ref[...] * dis


_scale = pl.pallas_call(
    _scale_body,
    grid=(NP // BR,),
    in_specs=[
        pl.BlockSpec((BR, 128), lambda i: (i, 0)),
        pl.BlockSpec((NC, BR, 1), lambda i: (0, i, 0)),
    ],
    out_specs=[
        pl.BlockSpec((BR, 128), lambda i: (i, 0)),
        pl.BlockSpec((BR, 1), lambda i: (i, 0)),
    ],
    out_shape=[
        jax.ShapeDtypeStruct((NP, 128), jnp.float32),
        jax.ShapeDtypeStruct((NP, 1), jnp.float32),
    ],
)


def _mid_body(p_ref, u_ref, d_ref, w1_ref, b1_ref, w2_ref, o_ref):
    a = (u_ref[...] + p_ref[0] + p_ref[1]) * d_ref[...]
    y = jnp.dot(a, w1_ref[...], preferred_element_type=jnp.float32)
    y = jnp.maximum(y + b1_ref[...], 0.0)
    o = jnp.dot(y, w2_ref[...], preferred_element_type=jnp.float32)
    o_ref[...] = o * d_ref[...]


_mid = pl.pallas_call(
    _mid_body,
    grid=(NP // BR,),
    in_specs=[
        pl.BlockSpec((NC, BR, 128), lambda i: (0, i, 0)),
        pl.BlockSpec((BR, 128), lambda i: (i, 0)),
        pl.BlockSpec((BR, 1), lambda i: (i, 0)),
        pl.BlockSpec((128, 256), lambda i: (0, 0)),
        pl.BlockSpec((1, 256), lambda i: (0, 0)),
        pl.BlockSpec((256, 128), lambda i: (0, 0)),
    ],
    out_specs=pl.BlockSpec((BR, 128), lambda i: (i, 0)),
    out_shape=jax.ShapeDtypeStruct((NP, 128), jnp.float32),
)


def _fin_body(p_ref, u_ref, d_ref, b2_ref, o_ref):
    s = (u_ref[...] + p_ref[0] + p_ref[1]) * d_ref[...]
    o_ref[...] = jnp.maximum(s + b2_ref[...], 0.0)


_fin = pl.pallas_call(
    _fin_body,
    grid=(NP // BR,),
    in_specs=[
        pl.BlockSpec((NC, BR, 128), lambda i: (0, i, 0)),
        pl.BlockSpec((BR, 128), lambda i: (i, 0)),
        pl.BlockSpec((BR, 1), lambda i: (i, 0)),
        pl.BlockSpec((1, 128), lambda i: (0, 0)),
    ],
    out_specs=pl.BlockSpec((BR, 128), lambda i: (i, 0)),
    out_shape=jax.ShapeDtypeStruct((NP, 128), jnp.float32),
)


def kernel(x, edge_index, W1, b1, W2, b2):
    ei = edge_index.astype(jnp.int32)
    dstd = ei[1].reshape(NC * NS, NCH, EK)
    pad = jnp.full((EPAD - E,), NP - 1, jnp.int32)
    srcp = jnp.concatenate([ei[0], pad])
    dstp = jnp.concatenate([ei[1], pad])
    pk = ((dstp << 16) | srcp).reshape(NC * NS, APR, 128)
    xp = jnp.pad(x, ((0, NP - N), (0, 0)))

    agg = _make_agg()
    deg = _make_deg()(dstd)
    u1, dis2 = _scale(xp, deg.reshape(NC, NP, 1))
    p1 = agg(u1, pk)
    u2 = _mid(p1, u1, dis2, W1, b1.reshape(1, -1), W2)
    p2 = agg(u2, pk)
    y = _fin(p2, u2, dis2, b2.reshape(1, -1))
    return y[:N]


# speedup vs baseline: 2.8882x; 2.8882x over previous
import functools

import jax
import jax.numpy as jnp
from jax import lax
from jax.experimental import pallas as pl
from jax.experimental.pallas import tpu as pltpu
from jax.experimental.pallas import tpu_sc as plsc

N = 10000
E = 320000
NP = 10240
NC = 2
NS = 16
RPT = NP // NS
BR = 512

def _mesh():
    return plsc.VectorSubcoreMesh(
        core_axis_name="c", subcore_axis_name="s",
        num_cores=NC, num_subcores=NS)


@functools.cache
def _make_deg():
    return functools.partial(
        pl.kernel,
        out_type=jax.ShapeDtypeStruct((NC, NP), jnp.float32),
        mesh=_mesh(),
        scratch_types=[
            pltpu.VMEM((80, 128), jnp.int32),
            [pltpu.VMEM((128,), jnp.int32) for _ in range(3)],
            pltpu.VMEM((128,), jnp.float32),
            pltpu.VMEM((RPT,), jnp.float32),
            [pltpu.SemaphoreType.DMA for _ in range(3)],
            pltpu.VMEM_SHARED((NP,), jnp.float32),
        ],
    )(_deg_body)


def _deg_body(pk_hbm, deg_hbm, pkbuf, dbufs, ones, dbuf, dsem, dacc):
    cid = lax.axis_index("c")
    tid = lax.axis_index("s")
    wid = tid * NC + cid

    pltpu.sync_copy(pk_hbm.at[wid], pkbuf)

    def fill(i, _):
        dbuf[pl.ds(i * 16, 16)] = jnp.zeros((16,), jnp.float32)
        return 0
    lax.fori_loop(0, RPT // 16, fill, 0)
    for k in range(8):
        ones[pl.ds(k * 16, 16)] = jnp.ones((16,), jnp.float32)
    pltpu.sync_copy(dbuf, dacc.at[pl.ds(tid * RPT, RPT)])
    plsc.subcore_barrier()

    def unpack(c, t):
        for k in range(8):
            v = pkbuf[c, pl.ds(k * 16, 16)]
            dbufs[t][pl.ds(k * 16, 16)] = v >> 16

    def fire(j, t):
        pltpu.async_copy(ones, dacc.at[dbufs[t]], dsem[t], add=True)

    def drain(t):
        pltpu.make_async_copy(ones, dacc.at[dbufs[t]], dsem[t]).wait()

    for b in range(3):
        unpack(b, b)
        fire(b, b)

    def body(o, _):
        for b in range(3):
            j = o * 3 + b
            drain(b)
            unpack(j, b)
            fire(j, b)
        return 0
    lax.fori_loop(1, 26, body, 0)
    for j in (78, 79):
        b = j % 3
        drain(b)
        unpack(j, b)
        fire(j, b)
    for b in range(3):
        drain(b)
    plsc.subcore_barrier()

    pltpu.sync_copy(dacc.at[pl.ds(tid * RPT, RPT)], dbuf)
    pltpu.sync_copy(dbuf, deg_hbm.at[cid, pl.ds(tid * RPT, RPT)])


EPAD = 327680
APR = 80
ACH = 64
ANCH = 160
NBUF = 3


@functools.cache
def _make_agg():
    return functools.partial(
        pl.kernel,
        out_type=jax.ShapeDtypeStruct((NC, NP, 128), jnp.float32),
        mesh=_mesh(),
        scratch_types=[
            pltpu.VMEM((APR, 128), jnp.int32),
            [pltpu.VMEM((ACH,), jnp.int32) for _ in range(NBUF)],
            [pltpu.VMEM((ACH,), jnp.int32) for _ in range(NBUF)],
            [pltpu.VMEM((ACH, 128), jnp.float32) for _ in range(NBUF)],
            [pltpu.SemaphoreType.DMA for _ in range(NBUF)],
            [pltpu.SemaphoreType.DMA for _ in range(NBUF)],
            pltpu.VMEM_SHARED((NP, 128), jnp.float32),
        ],
    )(_agg_body)


def _agg_body(u_hbm, pk_hbm, out_hbm,
              pkbuf, sbufs, dbufs, rows, gsem, ssem, acc):
    cid = lax.axis_index("c")
    tid = lax.axis_index("s")
    wid = tid * NC + cid

    pltpu.sync_copy(pk_hbm.at[wid], pkbuf)

    def unpack(c, t):
        row = c // 2
        base = (c % 2) * 64
        for k in range(4):
            v = pkbuf[row, pl.ds(base + k * 16, 16)]
            sbufs[t][pl.ds(k * 16, 16)] = v & 0xFFFF
            dbufs[t][pl.ds(k * 16, 16)] = v >> 16

    def gather_start(j, b):
        pltpu.async_copy(u_hbm.at[sbufs[b]], rows[b], gsem[b])

    def gather_wait(b):
        pltpu.make_async_copy(u_hbm.at[sbufs[b]], rows[b], gsem[b]).wait()

    def scatter_start(j, b):
        pltpu.async_copy(rows[b], acc.at[dbufs[b]], ssem[b], add=True)

    def scatter_wait(b):
        pltpu.make_async_copy(rows[b], acc.at[dbufs[b]], ssem[b]).wait()

    def fill(i, _):
        rows[0][i // 8, pl.ds((i % 8) * 16, 16)] = jnp.zeros((16,), jnp.float32)
        return 0
    lax.fori_loop(0, ACH * 8, fill, 0)
    r0 = tid * RPT
    for k in range(RPT // ACH):
        pltpu.sync_copy(rows[0], acc.at[pl.ds(r0 + k * ACH, ACH)])
    unpack(0, 0)
    unpack(1, 1)
    gather_start(0, 0)
    gather_start(1, 1)
    plsc.subcore_barrier()

    gather_wait(0)
    scatter_start(0, 0)
    unpack(2, 2)
    gather_start(2, 2)
    gather_wait(1)
    scatter_wait(0)
    scatter_start(1, 1)
    unpack(3, 0)
    gather_start(3, 0)
    gather_wait(2)
    scatter_wait(1)
    scatter_start(2, 2)
    unpack(4, 1)
    gather_start(4, 1)

    def body(o, _):
        for b in range(NBUF):
            j = o * NBUF + b
            gather_wait(b)
            b2 = (b + 2) % NBUF
            scatter_wait(b2)
            scatter_start(j, b)
            unpack(j + 2, b2)
            gather_start(j + 2, b2)
        return 0
    lax.fori_loop(1, (ANCH - 4) // NBUF, body, 0)

    for j in range(ANCH - 4, ANCH):
        b = j % NBUF
        gather_wait(b)
        if j + 2 < ANCH:
            scatter_wait((j + 2) % NBUF)
        scatter_start(j, b)
        if j + 2 < ANCH:
            unpack(j + 2, (j + 2) % NBUF)
            gather_start(j + 2, (j + 2) % NBUF)
    scatter_wait((ANCH - 3) % NBUF)
    scatter_wait((ANCH - 2) % NBUF)
    scatter_wait((ANCH - 1) % NBUF)
    plsc.subcore_barrier()

    for k in range(RPT // ACH):
        pltpu.sync_copy(acc.at[pl.ds(r0 + k * ACH, ACH)], rows[0])
        pltpu.sync_copy(rows[0], out_hbm.at[cid, pl.ds(r0 + k * ACH, ACH)])


def _scale_body(x_ref, deg_ref, u_ref, dis_ref):
    deg = jnp.sum(deg_ref[...], axis=0)
    dis = lax.rsqrt(deg + 1.0)
    dis_ref[...] = dis
    u_ref[...] = x_ref[...] * dis


_scale = pl.pallas_call(
    _scale_body,
    grid=(NP // BR,),
    in_specs=[
        pl.BlockSpec((BR, 128), lambda i: (i, 0)),
        pl.BlockSpec((NC, BR, 1), lambda i: (0, i, 0)),
    ],
    out_specs=[
        pl.BlockSpec((BR, 128), lambda i: (i, 0)),
        pl.BlockSpec((BR, 1), lambda i: (i, 0)),
    ],
    out_shape=[
        jax.ShapeDtypeStruct((NP, 128), jnp.float32),
        jax.ShapeDtypeStruct((NP, 1), jnp.float32),
    ],
)


def _mid_body(p_ref, u_ref, d_ref, w1_ref, b1_ref, w2_ref, o_ref):
    a = (u_ref[...] + p_ref[0] + p_ref[1]) * d_ref[...]
    y = jnp.dot(a, w1_ref[...], preferred_element_type=jnp.float32)
    y = jnp.maximum(y + b1_ref[...], 0.0)
    o = jnp.dot(y, w2_ref[...], preferred_element_type=jnp.float32)
    o_ref[...] = o * d_ref[...]


_mid = pl.pallas_call(
    _mid_body,
    grid=(NP // BR,),
    in_specs=[
        pl.BlockSpec((NC, BR, 128), lambda i: (0, i, 0)),
        pl.BlockSpec((BR, 128), lambda i: (i, 0)),
        pl.BlockSpec((BR, 1), lambda i: (i, 0)),
        pl.BlockSpec((128, 256), lambda i: (0, 0)),
        pl.BlockSpec((1, 256), lambda i: (0, 0)),
        pl.BlockSpec((256, 128), lambda i: (0, 0)),
    ],
    out_specs=pl.BlockSpec((BR, 128), lambda i: (i, 0)),
    out_shape=jax.ShapeDtypeStruct((NP, 128), jnp.float32),
)


def _fin_body(p_ref, u_ref, d_ref, b2_ref, o_ref):
    s = (u_ref[...] + p_ref[0] + p_ref[1]) * d_ref[...]
    o_ref[...] = jnp.maximum(s + b2_ref[...], 0.0)


_fin = pl.pallas_call(
    _fin_body,
    grid=(NP // BR,),
    in_specs=[
        pl.BlockSpec((NC, BR, 128), lambda i: (0, i, 0)),
        pl.BlockSpec((BR, 128), lambda i: (i, 0)),
        pl.BlockSpec((BR, 1), lambda i: (i, 0)),
        pl.BlockSpec((1, 128), lambda i: (0, 0)),
    ],
    out_specs=pl.BlockSpec((BR, 128), lambda i: (i, 0)),
    out_shape=jax.ShapeDtypeStruct((NP, 128), jnp.float32),
)


def kernel(x, edge_index, W1, b1, W2, b2):
    ei = edge_index.astype(jnp.int32)
    pad = (jnp.arange(EPAD - E, dtype=jnp.int32) % (NP - N)) + N
    srcp = jnp.concatenate([ei[0], pad])
    dstp = jnp.concatenate([ei[1], pad])
    pk = ((dstp << 16) | srcp).reshape(NC * NS, APR, 128)
    xp = jnp.pad(x, ((0, NP - N), (0, 0)))

    agg = _make_agg()
    deg = _make_deg()(pk)
    u1, dis2 = _scale(xp, deg.reshape(NC, NP, 1))
    p1 = agg(u1, pk)
    u2 = _mid(p1, u1, dis2, W1, b1.reshape(1, -1), W2)
    p2 = agg(u2, pk)
    y = _fin(p2, u2, dis2, b2.reshape(1, -1))
    return y[:N]


# speedup vs baseline: 2.8953x; 1.0025x over previous
import functools

import jax
import jax.numpy as jnp
from jax import lax
from jax.experimental import pallas as pl
from jax.experimental.pallas import tpu as pltpu
from jax.experimental.pallas import tpu_sc as plsc

N = 10000
E = 320000
NP = 10240
NC = 2
NS = 16
RPT = NP // NS
BR = 512

def _mesh():
    return plsc.VectorSubcoreMesh(
        core_axis_name="c", subcore_axis_name="s",
        num_cores=NC, num_subcores=NS)


@functools.cache
def _make_deg():
    return functools.partial(
        pl.kernel,
        out_type=jax.ShapeDtypeStruct((NC, NP), jnp.float32),
        mesh=_mesh(),
        scratch_types=[
            pltpu.VMEM((80, 128), jnp.int32),
            [pltpu.VMEM((128,), jnp.int32) for _ in range(3)],
            pltpu.VMEM((128,), jnp.float32),
            pltpu.VMEM((RPT,), jnp.float32),
            [pltpu.SemaphoreType.DMA for _ in range(3)],
            pltpu.VMEM_SHARED((NP,), jnp.float32),
        ],
    )(_deg_body)


def _deg_body(pk_hbm, deg_hbm, pkbuf, dbufs, ones, dbuf, dsem, dacc):
    cid = lax.axis_index("c")
    tid = lax.axis_index("s")
    wid = tid * NC + cid

    pltpu.sync_copy(pk_hbm.at[wid], pkbuf)

    def fill(i, _):
        dbuf[pl.ds(i * 16, 16)] = jnp.zeros((16,), jnp.float32)
        return 0
    lax.fori_loop(0, RPT // 16, fill, 0)
    for k in range(8):
        ones[pl.ds(k * 16, 16)] = jnp.ones((16,), jnp.float32)
    pltpu.sync_copy(dbuf, dacc.at[pl.ds(tid * RPT, RPT)])
    plsc.subcore_barrier()

    def unpack(c, t):
        for k in range(8):
            v = pkbuf[c, pl.ds(k * 16, 16)]
            dbufs[t][pl.ds(k * 16, 16)] = v >> 16

    def fire(j, t):
        pltpu.async_copy(ones, dacc.at[dbufs[t]], dsem[t], add=True)

    def drain(t):
        pltpu.make_async_copy(ones, dacc.at[dbufs[t]], dsem[t]).wait()

    for b in range(3):
        unpack(b, b)
        fire(b, b)

    def body(o, _):
        for b in range(3):
            j = o * 3 + b
            drain(b)
            unpack(j, b)
            fire(j, b)
        return 0
    lax.fori_loop(1, 26, body, 0)
    for j in (78, 79):
        b = j % 3
        drain(b)
        unpack(j, b)
        fire(j, b)
    for b in range(3):
        drain(b)
    plsc.subcore_barrier()

    pltpu.sync_copy(dacc.at[pl.ds(tid * RPT, RPT)], dbuf)
    pltpu.sync_copy(dbuf, deg_hbm.at[cid, pl.ds(tid * RPT, RPT)])


EPAD = 327680
APR = 80
ACH = 64
ANCH = 160
NBUF = 3


@functools.cache
def _make_agg():
    return functools.partial(
        pl.kernel,
        out_type=jax.ShapeDtypeStruct((NC, NP, 128), jnp.float32),
        mesh=_mesh(),
        scratch_types=[
            pltpu.VMEM((APR, 128), jnp.int32),
            [pltpu.VMEM((ACH,), jnp.int32) for _ in range(NBUF)],
            [pltpu.VMEM((ACH,), jnp.int32) for _ in range(NBUF)],
            [pltpu.VMEM((ACH, 128), jnp.float32) for _ in range(NBUF)],
            [pltpu.SemaphoreType.DMA for _ in range(NBUF)],
            [pltpu.SemaphoreType.DMA for _ in range(NBUF)],
            pltpu.VMEM_SHARED((NP, 128), jnp.float32),
        ],
    )(_agg_body)


def _agg_body(u_hbm, pk_hbm, out_hbm,
              pkbuf, sbufs, dbufs, rows, gsem, ssem, acc):
    cid = lax.axis_index("c")
    tid = lax.axis_index("s")
    wid = tid * NC + cid

    pltpu.sync_copy(pk_hbm.at[wid], pkbuf)

    def unpack(c, t):
        row = c // 2
        base = (c % 2) * 64
        for k in range(4):
            v = pkbuf[row, pl.ds(base + k * 16, 16)]
            sbufs[t][pl.ds(k * 16, 16)] = v & 0xFFFF
            dbufs[t][pl.ds(k * 16, 16)] = v >> 16

    def gather_start(j, b):
        pltpu.async_copy(u_hbm.at[sbufs[b]], rows[b], gsem[b])

    def gather_wait(b):
        pltpu.make_async_copy(u_hbm.at[sbufs[b]], rows[b], gsem[b]).wait()

    def scatter_start(j, b):
        pltpu.async_copy(rows[b], acc.at[dbufs[b]], ssem[b], add=True)

    def scatter_wait(b):
        pltpu.make_async_copy(rows[b], acc.at[dbufs[b]], ssem[b]).wait()

    def fill(i, _):
        rows[0][i // 8, pl.ds((i % 8) * 16, 16)] = jnp.zeros((16,), jnp.float32)
        return 0
    lax.fori_loop(0, ACH * 8, fill, 0)
    r0 = tid * RPT
    for k in range(RPT // ACH):
        pltpu.sync_copy(rows[0], acc.at[pl.ds(r0 + k * ACH, ACH)])
    unpack(0, 0)
    unpack(1, 1)
    gather_start(0, 0)
    gather_start(1, 1)
    plsc.subcore_barrier()

    gather_wait(0)
    scatter_start(0, 0)
    unpack(2, 2)
    gather_start(2, 2)
    gather_wait(1)
    scatter_start(1, 1)
    scatter_wait(0)
    unpack(3, 0)
    gather_start(3, 0)
    gather_wait(2)
    scatter_start(2, 2)
    scatter_wait(1)
    unpack(4, 1)
    gather_start(4, 1)

    def body(o, _):
        for b in range(NBUF):
            j = o * NBUF + b
            gather_wait(b)
            scatter_start(j, b)
            b2 = (b + 2) % NBUF
            scatter_wait(b2)
            unpack(j + 2, b2)
            gather_start(j + 2, b2)
        return 0
    lax.fori_loop(1, (ANCH - 4) // NBUF, body, 0)

    for j in range(ANCH - 4, ANCH):
        b = j % NBUF
        gather_wait(b)
        scatter_start(j, b)
        if j + 2 < ANCH:
            scatter_wait((j + 2) % NBUF)
            unpack(j + 2, (j + 2) % NBUF)
            gather_start(j + 2, (j + 2) % NBUF)
    scatter_wait((ANCH - 3) % NBUF)
    scatter_wait((ANCH - 2) % NBUF)
    scatter_wait((ANCH - 1) % NBUF)
    plsc.subcore_barrier()

    for k in range(RPT // ACH):
        pltpu.sync_copy(acc.at[pl.ds(r0 + k * ACH, ACH)], rows[0])
        pltpu.sync_copy(rows[0], out_hbm.at[cid, pl.ds(r0 + k * ACH, ACH)])


def _scale_body(x_ref, deg_ref, u_ref, dis_ref):
    deg = jnp.sum(deg_ref[...], axis=0)
    dis = lax.rsqrt(deg + 1.0)
    dis_ref[...] = dis
    u_ref[...] = x_ref[...] * dis


_scale = pl.pallas_call(
    _scale_body,
    grid=(NP // BR,),
    in_specs=[
        pl.BlockSpec((BR, 128), lambda i: (i, 0)),
        pl.BlockSpec((NC, BR, 1), lambda i: (0, i, 0)),
    ],
    out_specs=[
        pl.BlockSpec((BR, 128), lambda i: (i, 0)),
        pl.BlockSpec((BR, 1), lambda i: (i, 0)),
    ],
    out_shape=[
        jax.ShapeDtypeStruct((NP, 128), jnp.float32),
        jax.ShapeDtypeStruct((NP, 1), jnp.float32),
    ],
)


def _mid_body(p_ref, u_ref, d_ref, w1_ref, b1_ref, w2_ref, o_ref):
    a = (u_ref[...] + p_ref[0] + p_ref[1]) * d_ref[...]
    y = jnp.dot(a, w1_ref[...], preferred_element_type=jnp.float32)
    y = jnp.maximum(y + b1_ref[...], 0.0)
    o = jnp.dot(y, w2_ref[...], preferred_element_type=jnp.float32)
    o_ref[...] = o * d_ref[...]


_mid = pl.pallas_call(
    _mid_body,
    grid=(NP // BR,),
    in_specs=[
        pl.BlockSpec((NC, BR, 128), lambda i: (0, i, 0)),
        pl.BlockSpec((BR, 128), lambda i: (i, 0)),
        pl.BlockSpec((BR, 1), lambda i: (i, 0)),
        pl.BlockSpec((128, 256), lambda i: (0, 0)),
        pl.BlockSpec((1, 256), lambda i: (0, 0)),
        pl.BlockSpec((256, 128), lambda i: (0, 0)),
    ],
    out_specs=pl.BlockSpec((BR, 128), lambda i: (i, 0)),
    out_shape=jax.ShapeDtypeStruct((NP, 128), jnp.float32),
)


def _fin_body(p_ref, u_ref, d_ref, b2_ref, o_ref):
    s = (u_ref[...] + p_ref[0] + p_ref[1]) * d_ref[...]
    o_ref[...] = jnp.maximum(s + b2_ref[...], 0.0)


_fin = pl.pallas_call(
    _fin_body,
    grid=(NP // BR,),
    in_specs=[
        pl.BlockSpec((NC, BR, 128), lambda i: (0, i, 0)),
        pl.BlockSpec((BR, 128), lambda i: (i, 0)),
        pl.BlockSpec((BR, 1), lambda i: (i, 0)),
        pl.BlockSpec((1, 128), lambda i: (0, 0)),
    ],
    out_specs=pl.BlockSpec((BR, 128), lambda i: (i, 0)),
    out_shape=jax.ShapeDtypeStruct((NP, 128), jnp.float32),
)


def kernel(x, edge_index, W1, b1, W2, b2):
    ei = edge_index.astype(jnp.int32)
    pad = (jnp.arange(EPAD - E, dtype=jnp.int32) % (NP - N)) + N
    srcp = jnp.concatenate([ei[0], pad])
    dstp = jnp.concatenate([ei[1], pad])
    pk = ((dstp << 16) | srcp).reshape(NC * NS, APR, 128)
    xp = jnp.pad(x, ((0, NP - N), (0, 0)))

    agg = _make_agg()
    deg = _make_deg()(pk)
    u1, dis2 = _scale(xp, deg.reshape(NC, NP, 1))
    p1 = agg(u1, pk)
    u2 = _mid(p1, u1, dis2, W1, b1.reshape(1, -1), W2)
    p2 = agg(u2, pk)
    y = _fin(p2, u2, dis2, b2.reshape(1, -1))
    return y[:N]


# speedup vs baseline: 2.9721x; 1.0265x over previous
import functools

import jax
import jax.numpy as jnp
from jax import lax
from jax.experimental import pallas as pl
from jax.experimental.pallas import tpu as pltpu
from jax.experimental.pallas import tpu_sc as plsc

N = 10000
E = 320000
NP = 10240
NC = 2
NS = 16
RPT = NP // NS
BR = 512

def _mesh():
    return plsc.VectorSubcoreMesh(
        core_axis_name="c", subcore_axis_name="s",
        num_cores=NC, num_subcores=NS)


@functools.cache
def _make_deg():
    return functools.partial(
        pl.kernel,
        out_type=jax.ShapeDtypeStruct((NC, NP), jnp.float32),
        mesh=_mesh(),
        scratch_types=[
            pltpu.VMEM((80, 128), jnp.int32),
            [pltpu.VMEM((128,), jnp.int32) for _ in range(3)],
            pltpu.VMEM((128,), jnp.float32),
            pltpu.VMEM((RPT,), jnp.float32),
            [pltpu.SemaphoreType.DMA for _ in range(3)],
            pltpu.VMEM_SHARED((NP,), jnp.float32),
        ],
    )(_deg_body)


def _deg_body(pk_hbm, deg_hbm, pkbuf, dbufs, ones, dbuf, dsem, dacc):
    cid = lax.axis_index("c")
    tid = lax.axis_index("s")
    wid = tid * NC + cid

    pltpu.sync_copy(pk_hbm.at[wid], pkbuf)

    def fill(i, _):
        dbuf[pl.ds(i * 16, 16)] = jnp.zeros((16,), jnp.float32)
        return 0
    lax.fori_loop(0, RPT // 16, fill, 0)
    for k in range(8):
        ones[pl.ds(k * 16, 16)] = jnp.ones((16,), jnp.float32)
    pltpu.sync_copy(dbuf, dacc.at[pl.ds(tid * RPT, RPT)])
    plsc.subcore_barrier()

    def unpack(c, t):
        for k in range(8):
            v = pkbuf[c, pl.ds(k * 16, 16)]
            dbufs[t][pl.ds(k * 16, 16)] = v >> 16

    def fire(j, t):
        pltpu.async_copy(ones, dacc.at[dbufs[t]], dsem[t], add=True)

    def drain(t):
        pltpu.make_async_copy(ones, dacc.at[dbufs[t]], dsem[t]).wait()

    for b in range(3):
        unpack(b, b)
        fire(b, b)

    def body(o, _):
        for b in range(3):
            j = o * 3 + b
            drain(b)
            unpack(j, b)
            fire(j, b)
        return 0
    lax.fori_loop(1, 26, body, 0)
    for j in (78, 79):
        b = j % 3
        drain(b)
        unpack(j, b)
        fire(j, b)
    for b in range(3):
        drain(b)
    plsc.subcore_barrier()

    pltpu.sync_copy(dacc.at[pl.ds(tid * RPT, RPT)], dbuf)
    pltpu.sync_copy(dbuf, deg_hbm.at[cid, pl.ds(tid * RPT, RPT)])


EPAD = 327680
APR = 80
ACH = 64
ANCH = 160
NBUF = 3


@functools.cache
def _make_agg():
    return functools.partial(
        pl.kernel,
        out_type=jax.ShapeDtypeStruct((NC, NP, 128), jnp.float32),
        mesh=_mesh(),
        scratch_types=[
            pltpu.VMEM((APR, 128), jnp.int32),
            [pltpu.VMEM((ACH,), jnp.int32) for _ in range(NBUF)],
            [pltpu.VMEM((ACH,), jnp.int32) for _ in range(NBUF)],
            [pltpu.VMEM((ACH, 128), jnp.float32) for _ in range(NBUF)],
            [pltpu.SemaphoreType.DMA for _ in range(NBUF)],
            [pltpu.SemaphoreType.DMA for _ in range(NBUF)],
            pltpu.VMEM_SHARED((NP, 128), jnp.float32),
        ],
    )(_agg_body)


def _agg_body(u_hbm, pk_hbm, out_hbm,
              pkbuf, sbufs, dbufs, rows, gsem, ssem, acc):
    cid = lax.axis_index("c")
    tid = lax.axis_index("s")
    wid = tid * NC + cid

    def unpack(c, t):
        row = c // 2
        base = (c % 2) * 64
        for k in range(4):
            v = pkbuf[row, pl.ds(base + k * 16, 16)]
            sbufs[t][pl.ds(k * 16, 16)] = v & 0xFFFF
            dbufs[t][pl.ds(k * 16, 16)] = v >> 16

    def gather_start(j, b):
        pltpu.async_copy(u_hbm.at[sbufs[b]], rows[b], gsem[b])

    def gather_wait(b):
        pltpu.make_async_copy(u_hbm.at[sbufs[b]], rows[b], gsem[b]).wait()

    def scatter_start(j, b):
        pltpu.async_copy(rows[b], acc.at[dbufs[b]], ssem[b], add=True)

    def scatter_wait(b):
        pltpu.make_async_copy(rows[b], acc.at[dbufs[b]], ssem[b]).wait()

    def fill(i, _):
        rows[0][i // 8, pl.ds((i % 8) * 16, 16)] = jnp.zeros((16,), jnp.float32)
        return 0
    lax.fori_loop(0, ACH * 8, fill, 0)
    r0 = tid * RPT
    for k in range(RPT // ACH):
        pltpu.async_copy(rows[0], acc.at[pl.ds(r0 + k * ACH, ACH)], ssem[0])
    pltpu.sync_copy(pk_hbm.at[wid], pkbuf)
    unpack(0, 0)
    unpack(1, 1)
    for k in range(RPT // ACH):
        pltpu.make_async_copy(rows[0], acc.at[pl.ds(r0, ACH)], ssem[0]).wait()
    gather_start(0, 0)
    gather_start(1, 1)
    plsc.subcore_barrier()

    gather_wait(0)
    scatter_start(0, 0)
    unpack(2, 2)
    gather_start(2, 2)
    gather_wait(1)
    scatter_start(1, 1)
    scatter_wait(0)
    unpack(3, 0)
    gather_start(3, 0)
    gather_wait(2)
    scatter_start(2, 2)
    scatter_wait(1)
    unpack(4, 1)
    gather_start(4, 1)

    def body(o, _):
        for b in range(NBUF):
            j = o * NBUF + b
            gather_wait(b)
            scatter_start(j, b)
            b2 = (b + 2) % NBUF
            scatter_wait(b2)
            unpack(j + 2, b2)
            gather_start(j + 2, b2)
        return 0
    lax.fori_loop(1, (ANCH - 4) // NBUF, body, 0)

    for j in range(ANCH - 4, ANCH):
        b = j % NBUF
        gather_wait(b)
        scatter_start(j, b)
        if j + 2 < ANCH:
            scatter_wait((j + 2) % NBUF)
            unpack(j + 2, (j + 2) % NBUF)
            gather_start(j + 2, (j + 2) % NBUF)
    scatter_wait((ANCH - 3) % NBUF)
    scatter_wait((ANCH - 2) % NBUF)
    scatter_wait((ANCH - 1) % NBUF)
    plsc.subcore_barrier()

    for k in range(RPT // ACH):
        b = k % NBUF
        if k >= NBUF:
            pltpu.make_async_copy(
                rows[b], out_hbm.at[cid, pl.ds(r0, ACH)], ssem[b]).wait()
        pltpu.async_copy(acc.at[pl.ds(r0 + k * ACH, ACH)], rows[b], gsem[b])
        pltpu.make_async_copy(
            acc.at[pl.ds(r0, ACH)], rows[b], gsem[b]).wait()
        pltpu.async_copy(
            rows[b], out_hbm.at[cid, pl.ds(r0 + k * ACH, ACH)], ssem[b])
    for k in range(RPT // ACH - NBUF, RPT // ACH):
        b = k % NBUF
        pltpu.make_async_copy(
            rows[b], out_hbm.at[cid, pl.ds(r0, ACH)], ssem[b]).wait()


def _scale_body(x_ref, deg_ref, u_ref, dis_ref):
    deg = jnp.sum(deg_ref[...], axis=0)
    dis = lax.rsqrt(deg + 1.0)
    dis_ref[...] = dis
    u_ref[...] = x_ref[...] * dis


_scale = pl.pallas_call(
    _scale_body,
    grid=(NP // BR,),
    in_specs=[
        pl.BlockSpec((BR, 128), lambda i: (i, 0)),
        pl.BlockSpec((NC, BR, 1), lambda i: (0, i, 0)),
    ],
    out_specs=[
        pl.BlockSpec((BR, 128), lambda i: (i, 0)),
        pl.BlockSpec((BR, 1), lambda i: (i, 0)),
    ],
    out_shape=[
        jax.ShapeDtypeStruct((NP, 128), jnp.float32),
        jax.ShapeDtypeStruct((NP, 1), jnp.float32),
    ],
)


def _mid_body(p_ref, u_ref, d_ref, w1_ref, b1_ref, w2_ref, o_ref):
    a = (u_ref[...] + p_ref[0] + p_ref[1]) * d_ref[...]
    y = jnp.dot(a, w1_ref[...], preferred_element_type=jnp.float32)
    y = jnp.maximum(y + b1_ref[...], 0.0)
    o = jnp.dot(y, w2_ref[...], preferred_element_type=jnp.float32)
    o_ref[...] = o * d_ref[...]


_mid = pl.pallas_call(
    _mid_body,
    grid=(NP // BR,),
    in_specs=[
        pl.BlockSpec((NC, BR, 128), lambda i: (0, i, 0)),
        pl.BlockSpec((BR, 128), lambda i: (i, 0)),
        pl.BlockSpec((BR, 1), lambda i: (i, 0)),
        pl.BlockSpec((128, 256), lambda i: (0, 0)),
        pl.BlockSpec((1, 256), lambda i: (0, 0)),
        pl.BlockSpec((256, 128), lambda i: (0, 0)),
    ],
    out_specs=pl.BlockSpec((BR, 128), lambda i: (i, 0)),
    out_shape=jax.ShapeDtypeStruct((NP, 128), jnp.float32),
)


def _fin_body(p_ref, u_ref, d_ref, b2_ref, o_ref):
    s = (u_ref[...] + p_ref[0] + p_ref[1]) * d_ref[...]
    o_ref[...] = jnp.maximum(s + b2_ref[...], 0.0)


_fin = pl.pallas_call(
    _fin_body,
    grid=(NP // BR,),
    in_specs=[
        pl.BlockSpec((NC, BR, 128), lambda i: (0, i, 0)),
        pl.BlockSpec((BR, 128), lambda i: (i, 0)),
        pl.BlockSpec((BR, 1), lambda i: (i, 0)),
        pl.BlockSpec((1, 128), lambda i: (0, 0)),
    ],
    out_specs=pl.BlockSpec((BR, 128), lambda i: (i, 0)),
    out_shape=jax.ShapeDtypeStruct((NP, 128), jnp.float32),
)


def kernel(x, edge_index, W1, b1, W2, b2):
    ei = edge_index.astype(jnp.int32)
    pad = (jnp.arange(EPAD - E, dtype=jnp.int32) % (NP - N)) + N
    srcp = jnp.concatenate([ei[0], pad])
    dstp = jnp.concatenate([ei[1], pad])
    pk = ((dstp << 16) | srcp).reshape(NC * NS, APR, 128)
    xp = jnp.pad(x, ((0, NP - N), (0, 0)))

    agg = _make_agg()
    deg = _make_deg()(pk)
    u1, dis2 = _scale(xp, deg.reshape(NC, NP, 1))
    p1 = agg(u1, pk)
    u2 = _mid(p1, u1, dis2, W1, b1.reshape(1, -1), W2)
    p2 = agg(u2, pk)
    y = _fin(p2, u2, dis2, b2.reshape(1, -1))
    return y[:N]
